# baseline (speedup 1.0000x reference)
import jax
import jax.numpy as jnp
from jax import lax
from jax.experimental import pallas as pl
from jax.experimental.pallas import tpu as pltpu
from jax.experimental.pallas import tpu_sc as plsc

_N = 10000
_E = 320000
_D = 128

_NC = 2
_NS = 16
_NW = _NC * _NS

_CHUNK = 80
_NCH = _E // _CHUNK
_CPW = _NCH // _NW
_NB = 3
_NPAD = 10240
_RPT = _NPAD // _NS


def _make_seg_sum(with_counts):
    def body(feat_hbm, el_hbm, zeros_hbm, *rest):
        if with_counts:
            (z1_hbm, out_hbm, cnt_hbm, acc, idx0, idx1, idx2,
             rows0, rows1, rows2, cnt_v,
             g0, g1, g2, s0, s1, s2, i0, i1, i2) = rest
        else:
            (out_hbm, acc, idx0, idx1, idx2,
             rows0, rows1, rows2,
             g0, g1, g2, s0, s1, s2, i0, i1, i2) = rest
        idx = (idx0, idx1, idx2)
        rows = (rows0, rows1, rows2)
        gsem = (g0, g1, g2)
        ssem = (s0, s1, s2)
        isem = (i0, i1, i2)
        c = lax.axis_index("c")
        s = lax.axis_index("s")
        wid = s * _NC + c
        r0 = s * _RPT
        pltpu.sync_copy(zeros_hbm.at[pl.ds(r0, _RPT)], acc.at[pl.ds(r0, _RPT)])
        if with_counts:
            pltpu.sync_copy(z1_hbm, cnt_v)
        plsc.subcore_barrier()
        ch0 = wid * _CPW

        def gather_start(b, ch):
            pltpu.async_copy(feat_hbm.at[idx[b].at[0]], rows[b], gsem[b])

        def gather_wait(b):
            pltpu.make_async_copy(
                feat_hbm.at[idx[b].at[0]], rows[b], gsem[b]).wait()

        def scatter_start(b):
            pltpu.async_copy(rows[b], acc.at[idx[b].at[1]], ssem[b], add=True)

        def scatter_wait(b):
            pltpu.make_async_copy(
                rows[b], acc.at[idx[b].at[1]], ssem[b]).wait()

        def counts(b):
            if with_counts:
                ones = jnp.ones((16,), jnp.float32)
                for j in range(_CHUNK // 16):
                    plsc.addupdate_scatter(
                        cnt_v, [idx[b][1, pl.ds(j * 16, 16)]], ones)

        pltpu.sync_copy(el_hbm.at[ch0], idx[0])
        gather_start(0, ch0)
        for t in (0, 1):
            b, pb = t, t + 1
            pltpu.sync_copy(el_hbm.at[ch0 + pb], idx[pb])
            gather_wait(b)
            counts(b)
            scatter_start(b)
            gather_start(pb, ch0 + pb)

        @pl.loop(2, _CPW - 3, step=3)
        def _steady(t0):
            for k in range(3):
                b = (2 + k) % 3
                pb = (b + 1) % 3
                t = t0 + k
                scatter_wait(pb)
                pltpu.async_copy(el_hbm.at[ch0 + t + 1], idx[pb], isem[pb])
                gather_wait(b)
                counts(b)
                scatter_start(b)
                pltpu.make_async_copy(
                    el_hbm.at[ch0 + t + 1], idx[pb], isem[pb]).wait()
                gather_start(pb, ch0 + t + 1)

        for t in (_CPW - 3, _CPW - 2):
            b = t % 3
            pb = (b + 1) % 3
            scatter_wait(pb)
            pltpu.sync_copy(el_hbm.at[ch0 + t + 1], idx[pb])
            gather_wait(b)
            counts(b)
            scatter_start(b)
            gather_start(pb, ch0 + t + 1)
        b = (_CPW - 1) % 3
        gather_wait(b)
        counts(b)
        scatter_start(b)
        for k in range(3):
            scatter_wait((b + 1 + k) % 3)

        plsc.subcore_barrier()
        pltpu.sync_copy(acc.at[pl.ds(r0, _RPT)], out_hbm.at[c, pl.ds(r0, _RPT)])
        if with_counts:
            pltpu.sync_copy(cnt_v, cnt_hbm.at[c, s])

    out_type = [jax.ShapeDtypeStruct((_NC, _NPAD, _D), jnp.float32)]
    scratch = [pltpu.VMEM_SHARED((_NPAD, _D), jnp.float32)]
    scratch += [pltpu.VMEM((2, _CHUNK), jnp.int32) for _ in range(_NB)]
    scratch += [pltpu.VMEM((_CHUNK, _D), jnp.float32) for _ in range(_NB)]
    if with_counts:
        out_type.append(jax.ShapeDtypeStruct((_NC, _NS, _NPAD), jnp.float32))
        scratch.append(pltpu.VMEM((_NPAD,), jnp.float32))
    scratch += [pltpu.SemaphoreType.DMA for _ in range(3 * _NB)]

    mesh = plsc.VectorSubcoreMesh(core_axis_name="c", subcore_axis_name="s")
    return pl.kernel(
        body, out_type=out_type, mesh=mesh, scratch_types=scratch,
        compiler_params=pltpu.CompilerParams(needs_layout_passes=False),
    )


_seg_cnt = _make_seg_sum(True)
_seg = _make_seg_sum(False)


def _layer1_body(acc_ref, cnt_ref, W_ref, b_ref, g_ref, bt_ref,
                 h_ref, dinv_ref):
    s = acc_ref[0, :_N, :] + acc_ref[1, :_N, :]
    cnt = jnp.sum(cnt_ref[:_N, :], axis=1, keepdims=True)
    dinv = 1.0 / jnp.maximum(cnt, 1.0)
    agg = s * dinv
    h = jnp.dot(agg, W_ref[...], preferred_element_type=jnp.float32) + b_ref[...]
    h = jnp.maximum(h, 0.0)
    m = jnp.mean(h, axis=-1, keepdims=True)
    d = h - m
    v = jnp.mean(d * d, axis=-1, keepdims=True)
    h_ref[...] = d * lax.rsqrt(v + 1e-5) * g_ref[...] + bt_ref[...]
    dinv_ref[...] = jnp.broadcast_to(dinv, (_N, _D))


def _layer2_body(acc_ref, dinv_ref, W_ref, b_ref, g_ref, bt_ref,
                 Wo_ref, bo_ref, out_ref):
    s = acc_ref[0, :_N, :] + acc_ref[1, :_N, :]
    agg = s * dinv_ref[...]
    h = jnp.dot(agg, W_ref[...], preferred_element_type=jnp.float32) + b_ref[...]
    h = jnp.maximum(h, 0.0)
    m = jnp.mean(h, axis=-1, keepdims=True)
    d = h - m
    v = jnp.mean(d * d, axis=-1, keepdims=True)
    h = d * lax.rsqrt(v + 1e-5) * g_ref[...] + bt_ref[...]
    out_ref[...] = (
        jnp.dot(h, Wo_ref[...], preferred_element_type=jnp.float32) + bo_ref[...]
    )


_tc_layer1 = pl.pallas_call(
    _layer1_body,
    out_shape=[
        jax.ShapeDtypeStruct((_N, _D), jnp.float32),
        jax.ShapeDtypeStruct((_N, _D), jnp.float32),
    ],
)

_tc_layer2 = pl.pallas_call(
    _layer2_body,
    out_shape=jax.ShapeDtypeStruct((_N, _D), jnp.float32),
)


@jax.jit
def kernel(x, edge_index, batch, W1, b1, g1, bt1, W2, b2, g2, bt2, Wo, bo):
    del batch
    el = jnp.swapaxes(edge_index.reshape(2, _NCH, _CHUNK), 0, 1)
    z128 = jnp.zeros((_NPAD, _D), jnp.float32)
    z1 = jnp.zeros((_NPAD,), jnp.float32)

    acc1, cnt = _seg_cnt(x, el, z128, z1)
    cnt_t = cnt.reshape(_NW, _NPAD).T
    h1, dinv = _tc_layer1(acc1, cnt_t, W1, b1.reshape(1, _D),
                          g1.reshape(1, _D), bt1.reshape(1, _D))
    (acc2,) = _seg(h1, el, z128)
    return _tc_layer2(acc2, dinv, W2, b2.reshape(1, _D), g2.reshape(1, _D),
                      bt2.reshape(1, _D), Wo, bo.reshape(1, _D))


# speedup vs baseline: 1.4512x; 1.4512x over previous
import jax
import jax.numpy as jnp
from jax import lax
from jax.experimental import pallas as pl
from jax.experimental.pallas import tpu as pltpu
from jax.experimental.pallas import tpu_sc as plsc

_N = 10000
_E = 320000
_D = 128

_NC = 2
_NS = 16
_NW = _NC * _NS

_CHUNK = 128
_NCH = _E // _CHUNK
_CPW = _NCH // _NW
_XTRA = _NCH - _CPW * _NW
_NPAD = 10240
_RPT = _NPAD // _NS


def _make_seg_sum(with_counts):
    def body(feat_hbm, el_hbm, zeros_hbm, *rest):
        if with_counts:
            (z1_hbm, out_hbm, cnt_hbm, acc, idx0, idx1, idx2, idx3,
             rows0, rows1, cnt_v, g0, g1, i0, i1, i2, i3) = rest
        else:
            (out_hbm, acc, idx0, idx1, idx2, idx3,
             rows0, rows1, g0, g1, i0, i1, i2, i3) = rest
        idx = (idx0, idx1, idx2, idx3)
        rows = (rows0, rows1)
        gsem = (g0, g1)
        isem = (i0, i1, i2, i3)
        c = lax.axis_index("c")
        s = lax.axis_index("s")
        wid = s * _NC + c
        r0 = s * _RPT
        pltpu.sync_copy(zeros_hbm.at[pl.ds(r0, _RPT)], acc.at[pl.ds(r0, _RPT)])
        if with_counts:
            pltpu.sync_copy(z1_hbm, cnt_v)
        plsc.subcore_barrier()
        ch0 = wid * _CPW

        def gather_start(t4, t2, ch):
            pltpu.async_copy(feat_hbm.at[idx[t4].at[0]], rows[t2], gsem[t2])

        def consume(t4, t2):
            pltpu.make_async_copy(
                feat_hbm.at[idx[t4].at[0]], rows[t2], gsem[t2]).wait()
            if with_counts:
                ones = jnp.ones((16,), jnp.float32)
                for j in range(_CHUNK // 16):
                    plsc.addupdate_scatter(
                        cnt_v, [idx[t4][1, pl.ds(j * 16, 16)]], ones)
            pltpu.sync_copy(rows[t2], acc.at[idx[t4].at[1]], add=True)

        for b in (0, 1):
            pltpu.sync_copy(el_hbm.at[ch0 + b], idx[b])
            gather_start(b, b, ch0 + b)

        @pl.loop(0, _CPW - 2, step=4)
        def _steady(t0):
            for k in range(4):
                b2 = k % 2
                pb = (k + 2) % 4
                ch = ch0 + t0 + k + 2
                pltpu.async_copy(el_hbm.at[ch], idx[pb], isem[pb])
                consume(k, b2)
                pltpu.make_async_copy(el_hbm.at[ch], idx[pb], isem[pb]).wait()
                gather_start(pb, b2, ch)

        for b in (0, 1):
            consume(b, b)

        @pl.when(wid < _XTRA)
        def _extra():
            ch = _CPW * _NW + wid
            pltpu.sync_copy(el_hbm.at[ch], idx[0])
            gather_start(0, 0, ch)
            consume(0, 0)

        plsc.subcore_barrier()
        pltpu.sync_copy(acc.at[pl.ds(r0, _RPT)], out_hbm.at[c, pl.ds(r0, _RPT)])
        if with_counts:
            pltpu.sync_copy(cnt_v, cnt_hbm.at[c, s])

    out_type = [jax.ShapeDtypeStruct((_NC, _NPAD, _D), jnp.float32)]
    scratch = [pltpu.VMEM_SHARED((_NPAD, _D), jnp.float32)]
    scratch += [pltpu.VMEM((2, _CHUNK), jnp.int32) for _ in range(4)]
    scratch += [pltpu.VMEM((_CHUNK, _D), jnp.float32) for _ in range(2)]
    if with_counts:
        out_type.append(jax.ShapeDtypeStruct((_NC, _NS, _NPAD), jnp.float32))
        scratch.append(pltpu.VMEM((_NPAD,), jnp.float32))
    scratch += [pltpu.SemaphoreType.DMA for _ in range(6)]

    mesh = plsc.VectorSubcoreMesh(core_axis_name="c", subcore_axis_name="s")
    return pl.kernel(
        body, out_type=out_type, mesh=mesh, scratch_types=scratch,
        compiler_params=pltpu.CompilerParams(needs_layout_passes=False),
    )


_seg_cnt = _make_seg_sum(True)
_seg = _make_seg_sum(False)


def _layer1_body(acc_ref, cnt_ref, W_ref, b_ref, g_ref, bt_ref,
                 h_ref, dinv_ref):
    s = acc_ref[0, :_N, :] + acc_ref[1, :_N, :]
    cnt = jnp.sum(cnt_ref[:_N, :], axis=1, keepdims=True)
    dinv = 1.0 / jnp.maximum(cnt, 1.0)
    agg = s * dinv
    h = jnp.dot(agg, W_ref[...], preferred_element_type=jnp.float32) + b_ref[...]
    h = jnp.maximum(h, 0.0)
    m = jnp.mean(h, axis=-1, keepdims=True)
    d = h - m
    v = jnp.mean(d * d, axis=-1, keepdims=True)
    h_ref[...] = d * lax.rsqrt(v + 1e-5) * g_ref[...] + bt_ref[...]
    dinv_ref[...] = jnp.broadcast_to(dinv, (_N, _D))


def _layer2_body(acc_ref, dinv_ref, W_ref, b_ref, g_ref, bt_ref,
                 Wo_ref, bo_ref, out_ref):
    s = acc_ref[0, :_N, :] + acc_ref[1, :_N, :]
    agg = s * dinv_ref[...]
    h = jnp.dot(agg, W_ref[...], preferred_element_type=jnp.float32) + b_ref[...]
    h = jnp.maximum(h, 0.0)
    m = jnp.mean(h, axis=-1, keepdims=True)
    d = h - m
    v = jnp.mean(d * d, axis=-1, keepdims=True)
    h = d * lax.rsqrt(v + 1e-5) * g_ref[...] + bt_ref[...]
    out_ref[...] = (
        jnp.dot(h, Wo_ref[...], preferred_element_type=jnp.float32) + bo_ref[...]
    )


_tc_layer1 = pl.pallas_call(
    _layer1_body,
    out_shape=[
        jax.ShapeDtypeStruct((_N, _D), jnp.float32),
        jax.ShapeDtypeStruct((_N, _D), jnp.float32),
    ],
)

_tc_layer2 = pl.pallas_call(
    _layer2_body,
    out_shape=jax.ShapeDtypeStruct((_N, _D), jnp.float32),
)


@jax.jit
def kernel(x, edge_index, batch, W1, b1, g1, bt1, W2, b2, g2, bt2, Wo, bo):
    del batch
    el = jnp.swapaxes(edge_index.reshape(2, _NCH, _CHUNK), 0, 1)
    z128 = jnp.zeros((_NPAD, _D), jnp.float32)
    z1 = jnp.zeros((_NPAD,), jnp.float32)

    acc1, cnt = _seg_cnt(x, el, z128, z1)
    cnt_t = cnt.reshape(_NW, _NPAD).T
    h1, dinv = _tc_layer1(acc1, cnt_t, W1, b1.reshape(1, _D),
                          g1.reshape(1, _D), bt1.reshape(1, _D))
    (acc2,) = _seg(h1, el, z128)
    return _tc_layer2(acc2, dinv, W2, b2.reshape(1, _D), g2.reshape(1, _D),
                      bt2.reshape(1, _D), Wo, bo.reshape(1, _D))
